# baseline (speedup 1.0000x reference)
import functools

import jax
import jax.numpy as jnp
from jax import lax
from jax.experimental import pallas as pl
from jax.experimental.pallas import tpu as pltpu
from jax.experimental.pallas import tpu_sc as plsc

NUM_CLASSES_K = 100000
SMOOTHING_K = 0.1
CONFIDENCE_K = 1.0 - SMOOTHING_K
EPS_K = SMOOTHING_K / (NUM_CLASSES_K - 1)

CLASS_CHUNK = 4096


def _gather_pred_at_target(xt, target):
    info = plsc.get_sparse_core_info()
    nc, ns, nl = info.num_cores, info.num_subcores, info.num_lanes
    nw = nc * ns
    b = target.shape[0]
    d = xt.shape[1]
    b_per_w = b // nw
    mesh = plsc.VectorSubcoreMesh(core_axis_name="c", subcore_axis_name="s")

    @functools.partial(
        pl.kernel,
        out_type=jax.ShapeDtypeStruct((b,), jnp.float32),
        mesh=mesh,
        scratch_types=[
            pltpu.VMEM((b_per_w,), jnp.int32),
            pltpu.VMEM((b_per_w, d), jnp.float32),
            pltpu.VMEM((b_per_w,), jnp.float32),
            pltpu.SemaphoreType.DMA,
        ],
        compiler_params=pltpu.CompilerParams(
            needs_layout_passes=False, skip_device_barrier=True),
    )
    def sc_gather(table_hbm, idx_hbm, out_hbm, idx_v, rows_v, g_v, sem):
        wid = lax.axis_index("s") * nc + lax.axis_index("c")
        base = wid * b_per_w
        pltpu.sync_copy(idx_hbm.at[pl.ds(base, b_per_w)], idx_v)
        pltpu.async_copy(table_hbm.at[idx_v], rows_v, sem).wait()
        for v in range(b_per_w // nl):
            r_idx = lax.iota(jnp.int32, nl) + v * nl
            c_idx = r_idx + base
            g_v[pl.ds(v * nl, nl)] = plsc.load_gather(rows_v, [r_idx, c_idx])
        pltpu.sync_copy(g_v, out_hbm.at[pl.ds(base, b_per_w)])

    return sc_gather(xt, target)


def _colsum(x):
    ones = jnp.ones((1, x.shape[0]), jnp.float32)
    return jax.lax.dot_general(ones, x, (((1,), (0,)), ((), ())),
                               preferred_element_type=jnp.float32)


def _accumulate(x_ref, row0, m_ref, s_ref, t_ref, *, num_classes, masked):
    if masked:
        row = jax.lax.broadcasted_iota(jnp.int32, x_ref.shape, 0) + row0
        valid = row < num_classes
        x_max_in = jnp.where(valid, x_ref[...], -jnp.inf)
        x_sum_in = jnp.where(valid, x_ref[...], 0.0)
    else:
        x_max_in = x_ref[...]
        x_sum_in = x_ref[...]

    m_old = m_ref[...]
    chunk_max = jnp.max(x_max_in, axis=0, keepdims=True)
    m_new = jnp.maximum(m_old, chunk_max)
    s_ref[...] = s_ref[...] * jnp.exp(m_old - m_new) + _colsum(
        jnp.exp(x_max_in - m_new))
    m_ref[...] = m_new

    t_ref[...] = t_ref[...] + _colsum(x_sum_in)


def _loss_kernel(xt_ref, out_ref, m_ref, s_ref, t_ref,
                 *, num_chunks, num_classes, total_rows):
    j = pl.program_id(0)

    @pl.when(j == 0)
    def _init():
        m_ref[...] = jnp.full_like(m_ref, -jnp.inf)
        s_ref[...] = jnp.zeros_like(s_ref)
        t_ref[...] = jnp.zeros_like(t_ref)

    row0 = j * CLASS_CHUNK

    @pl.when(j < num_chunks - 1)
    def _full():
        _accumulate(xt_ref, row0, m_ref, s_ref, t_ref,
                    num_classes=num_classes, masked=False)

    @pl.when(j == num_chunks - 1)
    def _last():
        _accumulate(xt_ref, row0, m_ref, s_ref, t_ref,
                    num_classes=num_classes,
                    masked=(num_classes % CLASS_CHUNK != 0))
        lse = m_ref[...] + jnp.log(s_ref[...])
        part = (-EPS_K * (t_ref[...] - num_classes * lse)
                + (CONFIDENCE_K - EPS_K) * lse)
        out_ref[...] = jnp.sum(part).reshape(1, 1) / total_rows


def _combine_kernel(a_ref, g_ref, out_ref, *, total_rows):
    out_ref[...] = a_ref[...] - (CONFIDENCE_K - EPS_K) * (
        jnp.sum(g_ref[...]).reshape(1, 1) / total_rows)


def kernel(pred, target):
    rows, num_classes = pred.shape
    xt = pred.T
    num_chunks = pl.cdiv(num_classes, CLASS_CHUNK)

    g = _gather_pred_at_target(xt, target.astype(jnp.int32))
    g2d = g.reshape(1, rows)

    part = pl.pallas_call(
        functools.partial(_loss_kernel, num_chunks=num_chunks,
                          num_classes=num_classes, total_rows=rows),
        grid=(num_chunks,),
        in_specs=[
            pl.BlockSpec((CLASS_CHUNK, rows), lambda j: (j, 0)),
        ],
        out_specs=pl.BlockSpec((1, 1), lambda j: (0, 0)),
        out_shape=jax.ShapeDtypeStruct((1, 1), jnp.float32),
        scratch_shapes=[
            pltpu.VMEM((1, rows), jnp.float32),
            pltpu.VMEM((1, rows), jnp.float32),
            pltpu.VMEM((1, rows), jnp.float32),
        ],
        compiler_params=pltpu.CompilerParams(
            vmem_limit_bytes=100 * 1024 * 1024),
    )(xt)

    out = pl.pallas_call(
        functools.partial(_combine_kernel, total_rows=rows),
        in_specs=[
            pl.BlockSpec((1, 1), lambda: (0, 0)),
            pl.BlockSpec((1, rows), lambda: (0, 0)),
        ],
        out_specs=pl.BlockSpec((1, 1), lambda: (0, 0)),
        out_shape=jax.ShapeDtypeStruct((1, 1), jnp.float32),
    )(part, g2d)
    return out[0, 0]


# speedup vs baseline: 1.0327x; 1.0327x over previous
import functools

import jax
import jax.numpy as jnp
from jax import lax
from jax.experimental import pallas as pl
from jax.experimental.pallas import tpu as pltpu
from jax.experimental.pallas import tpu_sc as plsc

NUM_CLASSES_K = 100000
SMOOTHING_K = 0.1
CONFIDENCE_K = 1.0 - SMOOTHING_K
EPS_K = SMOOTHING_K / (NUM_CLASSES_K - 1)

CLASS_CHUNK = 5000


def _gather_pred_at_target(xt, target):
    info = plsc.get_sparse_core_info()
    nc, ns, nl = info.num_cores, info.num_subcores, info.num_lanes
    nw = nc * ns
    b = target.shape[0]
    d = xt.shape[1]
    b_per_w = b // nw
    mesh = plsc.VectorSubcoreMesh(core_axis_name="c", subcore_axis_name="s")

    @functools.partial(
        pl.kernel,
        out_type=jax.ShapeDtypeStruct((b,), jnp.float32),
        mesh=mesh,
        scratch_types=[
            pltpu.VMEM((b_per_w,), jnp.int32),
            pltpu.VMEM((b_per_w, d), jnp.float32),
            pltpu.VMEM((b_per_w,), jnp.float32),
            pltpu.SemaphoreType.DMA,
        ],
        compiler_params=pltpu.CompilerParams(
            needs_layout_passes=False, skip_device_barrier=True),
    )
    def sc_gather(table_hbm, idx_hbm, out_hbm, idx_v, rows_v, g_v, sem):
        wid = lax.axis_index("s") * nc + lax.axis_index("c")
        base = wid * b_per_w
        pltpu.sync_copy(idx_hbm.at[pl.ds(base, b_per_w)], idx_v)
        pltpu.async_copy(table_hbm.at[idx_v], rows_v, sem).wait()
        for v in range(b_per_w // nl):
            r_idx = lax.iota(jnp.int32, nl) + v * nl
            c_idx = r_idx + base
            g_v[pl.ds(v * nl, nl)] = plsc.load_gather(rows_v, [r_idx, c_idx])
        pltpu.sync_copy(g_v, out_hbm.at[pl.ds(base, b_per_w)])

    return sc_gather(xt, target)


def _colsum(x):
    ones = jnp.ones((1, x.shape[0]), jnp.float32)
    return jax.lax.dot_general(ones, x, (((1,), (0,)), ((), ())),
                               preferred_element_type=jnp.float32)


def _accumulate(x_ref, row0, m_ref, s_ref, t_ref, *, num_classes, masked):
    if masked:
        row = jax.lax.broadcasted_iota(jnp.int32, x_ref.shape, 0) + row0
        valid = row < num_classes
        x_max_in = jnp.where(valid, x_ref[...], -jnp.inf)
        x_sum_in = jnp.where(valid, x_ref[...], 0.0)
    else:
        x_max_in = x_ref[...]
        x_sum_in = x_ref[...]

    m_old = m_ref[...]
    chunk_max = jnp.max(x_max_in, axis=0, keepdims=True)
    m_new = jnp.maximum(m_old, chunk_max)
    s_ref[...] = s_ref[...] * jnp.exp(m_old - m_new) + _colsum(
        jnp.exp(x_max_in - m_new))
    m_ref[...] = m_new

    t_ref[...] = t_ref[...] + _colsum(x_sum_in)


def _loss_kernel(xt_ref, out_ref, m_ref, s_ref, t_ref,
                 *, num_chunks, num_classes, total_rows):
    j = pl.program_id(0)

    @pl.when(j == 0)
    def _init():
        m_ref[...] = jnp.full_like(m_ref, -jnp.inf)
        s_ref[...] = jnp.zeros_like(s_ref)
        t_ref[...] = jnp.zeros_like(t_ref)

    row0 = j * CLASS_CHUNK

    @pl.when(j < num_chunks - 1)
    def _full():
        _accumulate(xt_ref, row0, m_ref, s_ref, t_ref,
                    num_classes=num_classes, masked=False)

    @pl.when(j == num_chunks - 1)
    def _last():
        _accumulate(xt_ref, row0, m_ref, s_ref, t_ref,
                    num_classes=num_classes,
                    masked=(num_classes % CLASS_CHUNK != 0))
        lse = m_ref[...] + jnp.log(s_ref[...])
        part = (-EPS_K * (t_ref[...] - num_classes * lse)
                + (CONFIDENCE_K - EPS_K) * lse)
        out_ref[...] = jnp.sum(part).reshape(1, 1) / total_rows


def _combine_kernel(a_ref, g_ref, out_ref, *, total_rows):
    out_ref[...] = a_ref[...] - (CONFIDENCE_K - EPS_K) * (
        jnp.sum(g_ref[...]).reshape(1, 1) / total_rows)


def kernel(pred, target):
    rows, num_classes = pred.shape
    xt = pred.T
    num_chunks = pl.cdiv(num_classes, CLASS_CHUNK)

    g = _gather_pred_at_target(xt, target.astype(jnp.int32))
    g2d = g.reshape(1, rows)

    part = pl.pallas_call(
        functools.partial(_loss_kernel, num_chunks=num_chunks,
                          num_classes=num_classes, total_rows=rows),
        grid=(num_chunks,),
        in_specs=[
            pl.BlockSpec((CLASS_CHUNK, rows), lambda j: (j, 0)),
        ],
        out_specs=pl.BlockSpec((1, 1), lambda j: (0, 0)),
        out_shape=jax.ShapeDtypeStruct((1, 1), jnp.float32),
        scratch_shapes=[
            pltpu.VMEM((1, rows), jnp.float32),
            pltpu.VMEM((1, rows), jnp.float32),
            pltpu.VMEM((1, rows), jnp.float32),
        ],
        compiler_params=pltpu.CompilerParams(
            vmem_limit_bytes=100 * 1024 * 1024),
    )(xt)

    out = pl.pallas_call(
        functools.partial(_combine_kernel, total_rows=rows),
        in_specs=[
            pl.BlockSpec((1, 1), lambda: (0, 0)),
            pl.BlockSpec((1, rows), lambda: (0, 0)),
        ],
        out_specs=pl.BlockSpec((1, 1), lambda: (0, 0)),
        out_shape=jax.ShapeDtypeStruct((1, 1), jnp.float32),
    )(part, g2d)
    return out[0, 0]


# speedup vs baseline: 1.0331x; 1.0004x over previous
import functools

import jax
import jax.numpy as jnp
from jax import lax
from jax.experimental import pallas as pl
from jax.experimental.pallas import tpu as pltpu
from jax.experimental.pallas import tpu_sc as plsc

NUM_CLASSES_K = 100000
SMOOTHING_K = 0.1
CONFIDENCE_K = 1.0 - SMOOTHING_K
EPS_K = SMOOTHING_K / (NUM_CLASSES_K - 1)

CLASS_CHUNK = 5000


def _gather_pred_at_target(xt, target):
    info = plsc.get_sparse_core_info()
    nc, ns, nl = info.num_cores, info.num_subcores, info.num_lanes
    nw = nc * ns
    b = target.shape[0]
    d = xt.shape[1]
    b_per_w = b // nw
    mesh = plsc.VectorSubcoreMesh(core_axis_name="c", subcore_axis_name="s")

    @functools.partial(
        pl.kernel,
        out_type=jax.ShapeDtypeStruct((b,), jnp.float32),
        mesh=mesh,
        scratch_types=[
            pltpu.VMEM((b_per_w,), jnp.int32),
            pltpu.VMEM((b_per_w, d), jnp.float32),
            pltpu.VMEM((b_per_w,), jnp.float32),
            pltpu.SemaphoreType.DMA,
        ],
        compiler_params=pltpu.CompilerParams(
            needs_layout_passes=False, skip_device_barrier=True),
    )
    def sc_gather(table_hbm, idx_hbm, out_hbm, idx_v, rows_v, g_v, sem):
        wid = lax.axis_index("s") * nc + lax.axis_index("c")
        base = wid * b_per_w
        pltpu.sync_copy(idx_hbm.at[pl.ds(base, b_per_w)], idx_v)
        pltpu.async_copy(table_hbm.at[idx_v], rows_v, sem).wait()
        for v in range(b_per_w // nl):
            r_idx = lax.iota(jnp.int32, nl) + v * nl
            c_idx = r_idx + base
            g_v[pl.ds(v * nl, nl)] = plsc.load_gather(rows_v, [r_idx, c_idx])
        pltpu.sync_copy(g_v, out_hbm.at[pl.ds(base, b_per_w)])

    return sc_gather(xt, target)


def _colsum(x):
    ones = jnp.ones((1, x.shape[0]), jnp.float32)
    return jax.lax.dot_general(ones, x, (((1,), (0,)), ((), ())),
                               preferred_element_type=jnp.float32)


def _accumulate(x_ref, row0, m_ref, s_ref, t_ref, *, num_classes, masked):
    if masked:
        row = jax.lax.broadcasted_iota(jnp.int32, x_ref.shape, 0) + row0
        valid = row < num_classes
        x_max_in = jnp.where(valid, x_ref[...], -jnp.inf)
        x_sum_in = jnp.where(valid, x_ref[...], 0.0)
    else:
        x_max_in = x_ref[...]
        x_sum_in = x_ref[...]

    m_old = m_ref[...]
    chunk_max = jnp.max(x_max_in, axis=0, keepdims=True)
    m_new = jnp.maximum(m_old, chunk_max)
    s_ref[...] = s_ref[...] * jnp.exp(m_old - m_new) + _colsum(
        jnp.exp(x_max_in - m_new))
    m_ref[...] = m_new

    t_ref[...] = t_ref[...] + _colsum(x_sum_in)


def _loss_kernel(xt_ref, out_ref, m_ref, s_ref, t_ref,
                 *, num_chunks, num_classes, total_rows):
    j = pl.program_id(0)

    @pl.when(j == 0)
    def _init():
        m_ref[...] = jnp.full_like(m_ref, -jnp.inf)
        s_ref[...] = jnp.zeros_like(s_ref)
        t_ref[...] = jnp.zeros_like(t_ref)

    row0 = j * CLASS_CHUNK
    ragged = num_classes % CLASS_CHUNK != 0

    if ragged:
        @pl.when(j < num_chunks - 1)
        def _full():
            _accumulate(xt_ref, row0, m_ref, s_ref, t_ref,
                        num_classes=num_classes, masked=False)

        @pl.when(j == num_chunks - 1)
        def _last_acc():
            _accumulate(xt_ref, row0, m_ref, s_ref, t_ref,
                        num_classes=num_classes, masked=True)
    else:
        _accumulate(xt_ref, row0, m_ref, s_ref, t_ref,
                    num_classes=num_classes, masked=False)

    @pl.when(j == num_chunks - 1)
    def _finalize():
        lse = m_ref[...] + jnp.log(s_ref[...])
        part = (-EPS_K * (t_ref[...] - num_classes * lse)
                + (CONFIDENCE_K - EPS_K) * lse)
        out_ref[...] = jnp.sum(part).reshape(1, 1) / total_rows


def _combine_kernel(a_ref, g_ref, out_ref, *, total_rows):
    out_ref[...] = a_ref[...] - (CONFIDENCE_K - EPS_K) * (
        jnp.sum(g_ref[...]).reshape(1, 1) / total_rows)


def kernel(pred, target):
    rows, num_classes = pred.shape
    xt = pred.T
    num_chunks = pl.cdiv(num_classes, CLASS_CHUNK)

    g = _gather_pred_at_target(xt, target.astype(jnp.int32))
    g2d = g.reshape(1, rows)

    part = pl.pallas_call(
        functools.partial(_loss_kernel, num_chunks=num_chunks,
                          num_classes=num_classes, total_rows=rows),
        grid=(num_chunks,),
        in_specs=[
            pl.BlockSpec((CLASS_CHUNK, rows), lambda j: (j, 0)),
        ],
        out_specs=pl.BlockSpec((1, 1), lambda j: (0, 0)),
        out_shape=jax.ShapeDtypeStruct((1, 1), jnp.float32),
        scratch_shapes=[
            pltpu.VMEM((1, rows), jnp.float32),
            pltpu.VMEM((1, rows), jnp.float32),
            pltpu.VMEM((1, rows), jnp.float32),
        ],
        compiler_params=pltpu.CompilerParams(
            vmem_limit_bytes=100 * 1024 * 1024),
    )(xt)

    out = pl.pallas_call(
        functools.partial(_combine_kernel, total_rows=rows),
        in_specs=[
            pl.BlockSpec((1, 1), lambda: (0, 0)),
            pl.BlockSpec((1, rows), lambda: (0, 0)),
        ],
        out_specs=pl.BlockSpec((1, 1), lambda: (0, 0)),
        out_shape=jax.ShapeDtypeStruct((1, 1), jnp.float32),
    )(part, g2d)
    return out[0, 0]
